# Initial kernel scaffold; baseline (speedup 1.0000x reference)
import jax
import jax.numpy as jnp
from jax.experimental import pallas as pl


def kernel(x, conv_w, conv_b, tree_w, tree_b, fc_w, fc_b):
    raise NotImplementedError("write your pallas kernel here")


# speedup vs baseline: 1.0581x; 1.0581x over previous
import jax
import jax.numpy as jnp
from jax.experimental import pallas as pl
from jax.experimental.pallas import tpu as pltpu

GROUPS = 3
F1 = 15
F2 = 16
PR = 7
BT = 64
SUB = 8
PLANE = 640
XL = 2 * GROUPS * PLANE
CHUNK = 256
FBLK = 448
KK = F1 * FBLK
NO = PR * F2


def _tree3_body(x_ref, cw_ref, cb_ref, tw_ref, tb_ref, fw_ref, fb_ref, o_ref,
                cbuf, pb0, pb1, pb2):
    pbs = (pb0, pb1, pb2)

    def conv_pool(j, carry):
        b0 = pl.multiple_of(j * SUB, SUB)
        for g in range(GROUPS):
            ebase = g * 2 * PLANE
            obase = ebase + PLANE
            for ch in range(2):
                base = ch * CHUNK
                te = {}
                to = {}
                for u in range(5):
                    for w in range(3):
                        off = base + u * 16 + w
                        te[(u, w)] = x_ref[pl.ds(b0, SUB), pl.ds(ebase + off, CHUNK)]
                        to[(u, w)] = x_ref[pl.ds(b0, SUB), pl.ds(obase + off, CHUNK)]
                for f in range(F1):
                    c = g * F1 + f
                    acc = te[(0, 0)] * cw_ref[c, 0]
                    for u in range(5):
                        for w in range(3):
                            if (u, w) != (0, 0):
                                acc = acc + te[(u, w)] * cw_ref[c, 5 * u + 2 * w]
                    for u in range(5):
                        for w in range(2):
                            acc = acc + to[(u, w)] * cw_ref[c, 5 * u + 2 * w + 1]
                    cbuf[f, :, pl.ds(base, CHUNK)] = acc
                    acc = to[(0, 0)] * cw_ref[c, 0]
                    for u in range(5):
                        for w in range(3):
                            if (u, w) != (0, 0):
                                acc = acc + to[(u, w)] * cw_ref[c, 5 * u + 2 * w]
                    for u in range(5):
                        for w in range(2):
                            acc = acc + te[(u, w + 1)] * cw_ref[c, 5 * u + 2 * w + 1]
                    cbuf[f, :, pl.ds(512 + base, CHUNK)] = acc
            for f in range(F1):
                c = g * F1 + f
                m = jnp.maximum(
                    jnp.maximum(cbuf[f, :, 0:448], cbuf[f, :, 16:464]),
                    jnp.maximum(cbuf[f, :, 512:960], cbuf[f, :, 528:976]))
                a = jax.nn.sigmoid(m + cb_ref[c])
                pbs[g][pl.ds(b0, SUB), pl.ds(f * FBLK, FBLK)] = a
        return carry

    jax.lax.fori_loop(0, BT // SUB, conv_pool, 0)

    z = fb_ref[...]
    for g in range(GROUPS):
        lhs = pbs[g][...].astype(jnp.bfloat16)
        y = jax.lax.dot_general(lhs, tw_ref[g], (((1,), (0,)), ((), ())),
                                preferred_element_type=jnp.float32)
        y = jax.nn.sigmoid(y + tb_ref[g])
        z = z + jnp.dot(y, fw_ref[g], preferred_element_type=jnp.float32)
    o_ref[...] = z


def _prep(x, conv_w, conv_b, tree_w, tree_b, fc_w, fc_b):
    n = x.shape[0]
    npad = -(-n // BT) * BT
    xr = x.astype(jnp.float32).reshape(n, GROUPS, 32, 16, 2)
    xr = jnp.transpose(xr, (0, 1, 4, 2, 3)).reshape(n, GROUPS, 2, 512)
    xr = jnp.pad(xr, ((0, npad - n), (0, 0), (0, 0), (0, PLANE - 512)))
    xf = xr.reshape(npad, XL)

    cw = conv_w.reshape(GROUPS * F1, 25).astype(jnp.float32)
    cb = conv_b.astype(jnp.float32)

    t = tree_w[0]
    t = jnp.transpose(t, (2, 1, 3, 5, 4, 6, 0))
    t = t.reshape(GROUPS, F1, PR, 2, 14, F2)
    w7 = jnp.zeros((GROUPS, F1, PR, 2, 32, PR, F2), jnp.float32)
    for p in range(PR):
        w7 = w7.at[:, :, p, :, 0:14, p, :].set(t[:, :, p])
    tw = w7.reshape(GROUPS, KK, NO).astype(jnp.bfloat16)

    tb = jnp.transpose(tree_b[0], (1, 2, 0)).reshape(GROUPS, 1, NO)
    tb = tb.astype(jnp.float32)
    fw = jnp.transpose(fc_w.reshape(10, F2, GROUPS, PR), (2, 3, 1, 0))
    fw = fw.reshape(GROUPS, NO, 10).astype(jnp.float32)
    fb = fc_b.reshape(1, 10).astype(jnp.float32)

    return xf, cw, cb, tw, tb, fw, fb, n, npad


def kernel(x, conv_w, conv_b, tree_w, tree_b, fc_w, fc_b):
    xf, cw, cb, tw, tb, fw, fb, n, npad = _prep(
        x, conv_w, conv_b, tree_w, tree_b, fc_w, fc_b)
    grid = (npad // BT,)
    flops_img = 2 * (F1 * GROUPS * 25 * 1024 + GROUPS * KK * NO + GROUPS * NO * 10)
    out = pl.pallas_call(
        _tree3_body,
        out_shape=jax.ShapeDtypeStruct((npad, 10), jnp.float32),
        grid=grid,
        in_specs=[
            pl.BlockSpec((BT, XL), lambda i: (i, 0)),
            pl.BlockSpec(memory_space=pltpu.MemorySpace.SMEM),
            pl.BlockSpec(memory_space=pltpu.MemorySpace.SMEM),
            pl.BlockSpec((GROUPS, KK, NO), lambda i: (0, 0, 0)),
            pl.BlockSpec((GROUPS, 1, NO), lambda i: (0, 0, 0)),
            pl.BlockSpec((GROUPS, NO, 10), lambda i: (0, 0, 0)),
            pl.BlockSpec((1, 10), lambda i: (0, 0)),
        ],
        out_specs=pl.BlockSpec((BT, 10), lambda i: (i, 0)),
        scratch_shapes=[pltpu.VMEM((F1, SUB, 1024), jnp.float32),
                        pltpu.VMEM((BT, KK), jnp.float32),
                        pltpu.VMEM((BT, KK), jnp.float32),
                        pltpu.VMEM((BT, KK), jnp.float32)],
        compiler_params=pltpu.CompilerParams(dimension_semantics=("parallel",)),
        cost_estimate=pl.CostEstimate(
            flops=npad * flops_img,
            transcendentals=npad * (GROUPS * F1 * FBLK + GROUPS * NO),
            bytes_accessed=int(xf.size * 4 + npad * 40 + tw.size * 2 + 4096)),
    )(xf, cw, cb, tw, tb, fw, fb)
    return out[:n]
